# Initial kernel scaffold; baseline (speedup 1.0000x reference)
import jax
import jax.numpy as jnp
from jax.experimental import pallas as pl


def kernel(x, adjs, embedding_weight, W0, a0, cls_weight, cls_a, dense_W, dense_b):
    raise NotImplementedError("write your pallas kernel here")


# speedup vs baseline: 1.9802x; 1.9802x over previous
import functools

import jax
import jax.numpy as jnp
from jax.experimental import pallas as pl

ALPHA = 0.2
NEG_BIG = -9e15


def _leaky_relu(v):
    return jnp.where(v >= 0, v, ALPHA * v)


def _elu(v):
    return jnp.where(v > 0, v, jnp.exp(jnp.minimum(v, 0.0)) - 1.0)


def _gat_block_kernel(x_ref, adj_ref, emb_ref, w_ref, a1_ref, a2_ref, out_ref,
                      *, heads, hidden):
    xv = x_ref[0, 0, :]
    adj = adj_ref[0, :, :]
    feats = xv[:, None] * emb_ref[:, :]
    wh_all = jnp.dot(feats, w_ref[:, :], preferred_element_type=jnp.float32)
    code = None
    for h in range(heads):
        wh = wh_all[:, h * hidden:(h + 1) * hidden]
        e1 = jnp.sum(wh * a1_ref[h, :][None, :], axis=1)
        e2 = jnp.sum(wh * a2_ref[h, :][None, :], axis=1)
        e = _leaky_relu(e1[:, None] + e2[None, :])
        e = jnp.where(adj > 0, e, NEG_BIG)
        m = jnp.max(e, axis=1, keepdims=True)
        p = jnp.exp(e - m)
        s = jnp.sum(p, axis=1, keepdims=True)
        attn = p / s
        hp = jnp.dot(attn, wh, preferred_element_type=jnp.float32)
        hp = _elu(hp)
        gated = xv[:, None] * hp
        cpart = jnp.max(gated, axis=0)
        code = cpart if code is None else jnp.concatenate([code, cpart])
    out_ref[0, 0, :] = code


def _tail_kernel(c0_ref, c1_ref, cls_ref, ca1_ref, ca2_ref, dw_ref, db_ref,
                 out_ref):
    cls = cls_ref[0, :]
    q = jnp.sum(cls * ca1_ref[0, :])
    e0 = _leaky_relu(q + jnp.sum(cls * ca2_ref[0, :]))
    e1 = _leaky_relu(q + jnp.sum(c0_ref[:, :] * ca2_ref[0, :][None, :], axis=1))
    e2 = _leaky_relu(q + jnp.sum(c1_ref[:, :] * ca2_ref[0, :][None, :], axis=1))
    m = jnp.maximum(jnp.maximum(e0, e1), e2)
    p0 = jnp.exp(e0 - m)
    p1 = jnp.exp(e1 - m)
    p2 = jnp.exp(e2 - m)
    s = p0 + p1 + p2
    pooled = (p0[:, None] * cls[None, :] + p1[:, None] * c0_ref[:, :]
              + p2[:, None] * c1_ref[:, :]) / s[:, None]
    out = jnp.dot(pooled, dw_ref[:, :], preferred_element_type=jnp.float32)
    out_ref[:, :] = _elu(out + db_ref[0, :][None, :])


def kernel(x, adjs, embedding_weight, W0, a0, cls_weight, cls_a, dense_W,
           dense_b):
    k, b, n = x.shape
    heads, embed, hidden = W0.shape
    d = heads * hidden
    pen = dense_W.shape[1]

    x_r = x.reshape(k * b, 1, n)
    adj_r = adjs.reshape(k * b, n, n)
    w_cat = jnp.transpose(W0, (1, 0, 2)).reshape(embed, d)
    a1s = a0[:, :hidden, 0]
    a2s = a0[:, hidden:, 0]

    codes = pl.pallas_call(
        functools.partial(_gat_block_kernel, heads=heads, hidden=hidden),
        grid=(k * b,),
        in_specs=[
            pl.BlockSpec((1, 1, n), lambda i: (i, 0, 0)),
            pl.BlockSpec((1, n, n), lambda i: (i, 0, 0)),
            pl.BlockSpec((n, embed), lambda i: (0, 0)),
            pl.BlockSpec((embed, d), lambda i: (0, 0)),
            pl.BlockSpec((heads, hidden), lambda i: (0, 0)),
            pl.BlockSpec((heads, hidden), lambda i: (0, 0)),
        ],
        out_specs=pl.BlockSpec((1, 1, d), lambda i: (i, 0, 0)),
        out_shape=jax.ShapeDtypeStruct((k * b, 1, d), jnp.float32),
    )(x_r, adj_r, embedding_weight, w_cat, a1s, a2s)

    codes = codes.reshape(k, b, d)
    out = pl.pallas_call(
        _tail_kernel,
        out_shape=jax.ShapeDtypeStruct((b, pen), jnp.float32),
    )(codes[0], codes[1], cls_weight.reshape(1, d),
      cls_a[:d, 0].reshape(1, d), cls_a[d:, 0].reshape(1, d),
      dense_W, dense_b.reshape(1, pen))
    return out


# speedup vs baseline: 2.3562x; 1.1899x over previous
import functools

import jax
import jax.numpy as jnp
from jax.experimental import pallas as pl
from jax.experimental.pallas import tpu as pltpu

ALPHA = 0.2
NEG_BIG = -9e15


def _leaky_relu(v):
    return jnp.where(v >= 0, v, ALPHA * v)


def _elu(v):
    return jnp.where(v > 0, v, jnp.exp(jnp.minimum(v, 0.0)) - 1.0)


def _gat_block_kernel(x_ref, adj_ref, emb_ref, w_ref, a1_ref, a2_ref, out_ref,
                      *, heads, hidden):
    xv = x_ref[0, 0, :]
    adj = adj_ref[0, :, :]
    maskf = jnp.where(adj > 0, 1.0, 0.0)
    feats = xv[:, None] * emb_ref[:, :]
    wh_all = jnp.dot(feats, w_ref[:, :], preferred_element_type=jnp.float32)
    code = None
    for h in range(heads):
        wh = wh_all[:, h * hidden:(h + 1) * hidden]
        e1 = jnp.sum(wh * a1_ref[h, :][None, :], axis=1)
        e2 = jnp.sum(wh * a2_ref[h, :][None, :], axis=1)
        u = jnp.exp(e1)[:, None] * jnp.exp(e2)[None, :]
        ua = jnp.exp(ALPHA * e1)[:, None] * jnp.exp(ALPHA * e2)[None, :]
        p = jnp.where(u > 1.0, u, ua) * maskf
        s = jnp.sum(p, axis=1)
        hp = jnp.dot(p, wh, preferred_element_type=jnp.float32)
        hp = _elu(hp * (1.0 / s)[:, None])
        gated = xv[:, None] * hp
        cpart = jnp.max(gated, axis=0)
        code = cpart if code is None else jnp.concatenate([code, cpart])
    out_ref[0, 0, :] = code


def _tail_kernel(c0_ref, c1_ref, cls_ref, ca1_ref, ca2_ref, dw_ref, db_ref,
                 out_ref):
    cls = cls_ref[0, :]
    q = jnp.sum(cls * ca1_ref[0, :])
    e0 = _leaky_relu(q + jnp.sum(cls * ca2_ref[0, :]))
    e1 = _leaky_relu(q + jnp.sum(c0_ref[:, :] * ca2_ref[0, :][None, :], axis=1))
    e2 = _leaky_relu(q + jnp.sum(c1_ref[:, :] * ca2_ref[0, :][None, :], axis=1))
    m = jnp.maximum(jnp.maximum(e0, e1), e2)
    p0 = jnp.exp(e0 - m)
    p1 = jnp.exp(e1 - m)
    p2 = jnp.exp(e2 - m)
    s = p0 + p1 + p2
    pooled = (p0[:, None] * cls[None, :] + p1[:, None] * c0_ref[:, :]
              + p2[:, None] * c1_ref[:, :]) / s[:, None]
    out = jnp.dot(pooled, dw_ref[:, :], preferred_element_type=jnp.float32)
    out_ref[:, :] = _elu(out + db_ref[0, :][None, :])


def kernel(x, adjs, embedding_weight, W0, a0, cls_weight, cls_a, dense_W,
           dense_b):
    k, b, n = x.shape
    heads, embed, hidden = W0.shape
    d = heads * hidden
    pen = dense_W.shape[1]

    x_r = x.reshape(k * b, 1, n)
    adj_r = adjs.reshape(k * b, n, n)
    w_cat = jnp.transpose(W0, (1, 0, 2)).reshape(embed, d)
    a1s = a0[:, :hidden, 0]
    a2s = a0[:, hidden:, 0]

    codes = pl.pallas_call(
        functools.partial(_gat_block_kernel, heads=heads, hidden=hidden),
        grid=(k * b,),
        in_specs=[
            pl.BlockSpec((1, 1, n), lambda i: (i, 0, 0)),
            pl.BlockSpec((1, n, n), lambda i: (i, 0, 0)),
            pl.BlockSpec((n, embed), lambda i: (0, 0)),
            pl.BlockSpec((embed, d), lambda i: (0, 0)),
            pl.BlockSpec((heads, hidden), lambda i: (0, 0)),
            pl.BlockSpec((heads, hidden), lambda i: (0, 0)),
        ],
        out_specs=pl.BlockSpec((1, 1, d), lambda i: (i, 0, 0)),
        out_shape=jax.ShapeDtypeStruct((k * b, 1, d), jnp.float32),
        compiler_params=pltpu.CompilerParams(
            dimension_semantics=("parallel",)),
    )(x_r, adj_r, embedding_weight, w_cat, a1s, a2s)

    codes = codes.reshape(k, b, d)
    out = pl.pallas_call(
        _tail_kernel,
        out_shape=jax.ShapeDtypeStruct((b, pen), jnp.float32),
    )(codes[0], codes[1], cls_weight.reshape(1, d),
      cls_a[:d, 0].reshape(1, d), cls_a[d:, 0].reshape(1, d),
      dense_W, dense_b.reshape(1, pen))
    return out


# speedup vs baseline: 4.4499x; 1.8886x over previous
import functools

import jax
import jax.numpy as jnp
from jax.experimental import pallas as pl
from jax.experimental.pallas import tpu as pltpu

ALPHA = 0.2
NEG_BIG = -9e15


def _leaky_relu(v):
    return jnp.where(v >= 0, v, ALPHA * v)


def _elu(v):
    return jnp.where(v > 0, v, jnp.exp(jnp.minimum(v, 0.0)) - 1.0)


def _gat_block_kernel(x_ref, adj_ref, emb_ref, embt_ref, w_ref, v1_ref,
                      v2_ref, out_ref, *, heads, hidden):
    n = adj_ref.shape[1]
    xv = x_ref[0, 0, :]
    adj = adj_ref[0, :, :]
    maskf = jnp.where(adj > 0, 1.0, 0.0)
    feats = xv[:, None] * emb_ref[:, :]
    feats_t = embt_ref[:, :] * xv[None, :]
    wh_all = jnp.dot(feats, w_ref[:, :], preferred_element_type=jnp.float32)
    e1c = jnp.dot(feats, v1_ref[:, :], preferred_element_type=jnp.float32)
    e2r = jnp.dot(v2_ref[:, :], feats_t, preferred_element_type=jnp.float32)
    exp_e1 = jnp.exp(e1c)
    exp_a1 = jnp.exp(ALPHA * e1c)
    exp_e2 = jnp.exp(e2r)
    exp_a2 = jnp.exp(ALPHA * e2r)
    ones_col = jnp.ones((n, 1), dtype=jnp.float32)
    code = None
    for h in range(heads):
        u = exp_e1[:, h:h + 1] * exp_e2[h:h + 1, :]
        ua = exp_a1[:, h:h + 1] * exp_a2[h:h + 1, :]
        p = jnp.maximum(u, ua) * maskf
        wh1 = jnp.concatenate(
            [wh_all[:, h * hidden:(h + 1) * hidden], ones_col], axis=1)
        hs = jnp.dot(p, wh1, preferred_element_type=jnp.float32)
        inv = 1.0 / hs[:, hidden:hidden + 1]
        hp = _elu(hs[:, :hidden] * inv)
        gated = xv[:, None] * hp
        cpart = jnp.max(gated, axis=0)
        code = cpart if code is None else jnp.concatenate([code, cpart])
    out_ref[0, 0, :] = code


def _tail_kernel(c0_ref, c1_ref, cls_ref, ca1_ref, ca2_ref, dw_ref, db_ref,
                 out_ref):
    cls = cls_ref[0, :]
    q = jnp.sum(cls * ca1_ref[0, :])
    e0 = _leaky_relu(q + jnp.sum(cls * ca2_ref[0, :]))
    e1 = _leaky_relu(q + jnp.sum(c0_ref[:, :] * ca2_ref[0, :][None, :], axis=1))
    e2 = _leaky_relu(q + jnp.sum(c1_ref[:, :] * ca2_ref[0, :][None, :], axis=1))
    m = jnp.maximum(jnp.maximum(e0, e1), e2)
    p0 = jnp.exp(e0 - m)
    p1 = jnp.exp(e1 - m)
    p2 = jnp.exp(e2 - m)
    s = p0 + p1 + p2
    pooled = (p0[:, None] * cls[None, :] + p1[:, None] * c0_ref[:, :]
              + p2[:, None] * c1_ref[:, :]) / s[:, None]
    out = jnp.dot(pooled, dw_ref[:, :], preferred_element_type=jnp.float32)
    out_ref[:, :] = _elu(out + db_ref[0, :][None, :])


def kernel(x, adjs, embedding_weight, W0, a0, cls_weight, cls_a, dense_W,
           dense_b):
    k, b, n = x.shape
    heads, embed, hidden = W0.shape
    d = heads * hidden
    pen = dense_W.shape[1]

    x_r = x.reshape(k * b, 1, n)
    adj_r = adjs.reshape(k * b, n, n)
    w_cat = jnp.transpose(W0, (1, 0, 2)).reshape(embed, d)
    a1s = a0[:, :hidden, 0]
    a2s = a0[:, hidden:, 0]
    v1 = jnp.einsum('hef,hf->eh', W0, a1s)
    v2 = jnp.einsum('hef,hf->he', W0, a2s)
    emb_t = embedding_weight.T

    codes = pl.pallas_call(
        functools.partial(_gat_block_kernel, heads=heads, hidden=hidden),
        grid=(k * b,),
        in_specs=[
            pl.BlockSpec((1, 1, n), lambda i: (i, 0, 0)),
            pl.BlockSpec((1, n, n), lambda i: (i, 0, 0)),
            pl.BlockSpec((n, embed), lambda i: (0, 0)),
            pl.BlockSpec((embed, n), lambda i: (0, 0)),
            pl.BlockSpec((embed, d), lambda i: (0, 0)),
            pl.BlockSpec((embed, heads), lambda i: (0, 0)),
            pl.BlockSpec((heads, embed), lambda i: (0, 0)),
        ],
        out_specs=pl.BlockSpec((1, 1, d), lambda i: (i, 0, 0)),
        out_shape=jax.ShapeDtypeStruct((k * b, 1, d), jnp.float32),
        compiler_params=pltpu.CompilerParams(
            dimension_semantics=("parallel",)),
    )(x_r, adj_r, embedding_weight, emb_t, w_cat, v1, v2)

    codes = codes.reshape(k, b, d)
    out = pl.pallas_call(
        _tail_kernel,
        out_shape=jax.ShapeDtypeStruct((b, pen), jnp.float32),
    )(codes[0], codes[1], cls_weight.reshape(1, d),
      cls_a[:d, 0].reshape(1, d), cls_a[d:, 0].reshape(1, d),
      dense_W, dense_b.reshape(1, pen))
    return out


# speedup vs baseline: 5.2802x; 1.1866x over previous
import functools

import jax
import jax.numpy as jnp
from jax.experimental import pallas as pl
from jax.experimental.pallas import tpu as pltpu

ALPHA = 0.2
NEG_BIG = -9e15


def _leaky_relu(v):
    return jnp.where(v >= 0, v, ALPHA * v)


def _elu(v):
    return jnp.where(v > 0, v, jnp.exp(jnp.minimum(v, 0.0)) - 1.0)


def _gat_block_kernel(x_ref, adj_ref, emb_ref, embt_ref, w_ref, v1_ref,
                      v2_ref, out_ref, *, heads, hidden):
    n = adj_ref.shape[1]
    xv = x_ref[0, 0, :]
    adj = adj_ref[0, :, :]
    maskf = jnp.where(adj > 0, 1.0, 0.0)
    feats = xv[:, None] * emb_ref[:, :]
    feats_t = embt_ref[:, :] * xv[None, :]
    wh_all = jnp.dot(feats, w_ref[:, :], preferred_element_type=jnp.float32)
    e1c = jnp.dot(feats, v1_ref[:, :], preferred_element_type=jnp.float32)
    e2r = jnp.dot(v2_ref[:, :], feats_t, preferred_element_type=jnp.float32)
    rcol = jnp.exp((ALPHA - 1.0) * e1c)
    exp_e2 = jnp.exp(e2r)
    exp_a2 = jnp.exp(ALPHA * e2r)
    ones_col = jnp.ones((n, 1), dtype=jnp.float32)
    code = None
    for h in range(heads):
        ua = rcol[:, h:h + 1] * exp_a2[h:h + 1, :]
        p = jnp.maximum(exp_e2[h:h + 1, :], ua) * maskf
        wh1 = jnp.concatenate(
            [wh_all[:, h * hidden:(h + 1) * hidden], ones_col], axis=1)
        hs = jnp.dot(p, wh1, preferred_element_type=jnp.float32)
        inv = 1.0 / hs[:, hidden:hidden + 1]
        hp = _elu(hs[:, :hidden] * inv)
        gated = xv[:, None] * hp
        cpart = jnp.max(gated, axis=0)
        code = cpart if code is None else jnp.concatenate([code, cpart])
    out_ref[0, 0, :] = code


def _tail_kernel(c0_ref, c1_ref, cls_ref, ca1_ref, ca2_ref, dw_ref, db_ref,
                 out_ref):
    cls = cls_ref[0, :]
    q = jnp.sum(cls * ca1_ref[0, :])
    e0 = _leaky_relu(q + jnp.sum(cls * ca2_ref[0, :]))
    e1 = _leaky_relu(q + jnp.sum(c0_ref[:, :] * ca2_ref[0, :][None, :], axis=1))
    e2 = _leaky_relu(q + jnp.sum(c1_ref[:, :] * ca2_ref[0, :][None, :], axis=1))
    m = jnp.maximum(jnp.maximum(e0, e1), e2)
    p0 = jnp.exp(e0 - m)
    p1 = jnp.exp(e1 - m)
    p2 = jnp.exp(e2 - m)
    s = p0 + p1 + p2
    pooled = (p0[:, None] * cls[None, :] + p1[:, None] * c0_ref[:, :]
              + p2[:, None] * c1_ref[:, :]) / s[:, None]
    out = jnp.dot(pooled, dw_ref[:, :], preferred_element_type=jnp.float32)
    out_ref[:, :] = _elu(out + db_ref[0, :][None, :])


def kernel(x, adjs, embedding_weight, W0, a0, cls_weight, cls_a, dense_W,
           dense_b):
    k, b, n = x.shape
    heads, embed, hidden = W0.shape
    d = heads * hidden
    pen = dense_W.shape[1]

    x_r = x.reshape(k * b, 1, n)
    adj_r = adjs.reshape(k * b, n, n)
    w_cat = jnp.transpose(W0, (1, 0, 2)).reshape(embed, d)
    a1s = a0[:, :hidden, 0]
    a2s = a0[:, hidden:, 0]
    v1 = jnp.einsum('hef,hf->eh', W0, a1s)
    v2 = jnp.einsum('hef,hf->he', W0, a2s)
    emb_t = embedding_weight.T

    codes = pl.pallas_call(
        functools.partial(_gat_block_kernel, heads=heads, hidden=hidden),
        grid=(k * b,),
        in_specs=[
            pl.BlockSpec((1, 1, n), lambda i: (i, 0, 0)),
            pl.BlockSpec((1, n, n), lambda i: (i, 0, 0)),
            pl.BlockSpec((n, embed), lambda i: (0, 0)),
            pl.BlockSpec((embed, n), lambda i: (0, 0)),
            pl.BlockSpec((embed, d), lambda i: (0, 0)),
            pl.BlockSpec((embed, heads), lambda i: (0, 0)),
            pl.BlockSpec((heads, embed), lambda i: (0, 0)),
        ],
        out_specs=pl.BlockSpec((1, 1, d), lambda i: (i, 0, 0)),
        out_shape=jax.ShapeDtypeStruct((k * b, 1, d), jnp.float32),
        compiler_params=pltpu.CompilerParams(
            dimension_semantics=("parallel",)),
    )(x_r, adj_r, embedding_weight, emb_t, w_cat, v1, v2)

    codes = codes.reshape(k, b, d)
    out = pl.pallas_call(
        _tail_kernel,
        out_shape=jax.ShapeDtypeStruct((b, pen), jnp.float32),
    )(codes[0], codes[1], cls_weight.reshape(1, d),
      cls_a[:d, 0].reshape(1, d), cls_a[d:, 0].reshape(1, d),
      dense_W, dense_b.reshape(1, pen))
    return out


# speedup vs baseline: 6.0657x; 1.1488x over previous
import functools

import jax
import jax.numpy as jnp
from jax.experimental import pallas as pl
from jax.experimental.pallas import tpu as pltpu

ALPHA = 0.2
NEG_BIG = -9e15


def _leaky_relu(v):
    return jnp.where(v >= 0, v, ALPHA * v)


def _elu(v):
    return jnp.where(v > 0, v, jnp.exp(jnp.minimum(v, 0.0)) - 1.0)


def _gat_block_kernel(x_ref, adj_ref, emb_ref, embt_ref, w_ref, v1_ref,
                      v2_ref, out_ref, *, heads, hidden):
    n = adj_ref.shape[1]
    xv = x_ref[0, 0, :]
    adj = adj_ref[0, :, :]
    maskb = jnp.where(adj > 0, 1.0, 0.0).astype(jnp.bfloat16)
    feats = xv[:, None] * emb_ref[:, :]
    feats_t = embt_ref[:, :] * xv[None, :]
    wh_all = jnp.dot(feats, w_ref[:, :], preferred_element_type=jnp.float32)
    e1c = jnp.dot(feats, v1_ref[:, :], preferred_element_type=jnp.float32)
    e2r = jnp.dot(v2_ref[:, :], feats_t, preferred_element_type=jnp.float32)
    rcol = jnp.exp((ALPHA - 1.0) * e1c).astype(jnp.bfloat16)
    exp_e2 = jnp.exp(e2r).astype(jnp.bfloat16)
    exp_a2 = jnp.exp(ALPHA * e2r).astype(jnp.bfloat16)
    ones_col = jnp.ones((n, 1), dtype=jnp.bfloat16)
    code = None
    for h in range(heads):
        ua = rcol[:, h:h + 1] * exp_a2[h:h + 1, :]
        p = jnp.maximum(exp_e2[h:h + 1, :], ua) * maskb
        wh1 = jnp.concatenate(
            [wh_all[:, h * hidden:(h + 1) * hidden].astype(jnp.bfloat16),
             ones_col], axis=1)
        hs = jnp.dot(p, wh1, preferred_element_type=jnp.float32)
        inv = 1.0 / hs[:, hidden:hidden + 1]
        hp = _elu(hs[:, :hidden] * inv)
        gated = xv[:, None] * hp
        cpart = jnp.max(gated, axis=0)
        code = cpart if code is None else jnp.concatenate([code, cpart])
    out_ref[0, 0, :] = code


def _tail_kernel(c0_ref, c1_ref, cls_ref, ca1_ref, ca2_ref, dw_ref, db_ref,
                 out_ref):
    cls = cls_ref[0, :]
    q = jnp.sum(cls * ca1_ref[0, :])
    e0 = _leaky_relu(q + jnp.sum(cls * ca2_ref[0, :]))
    e1 = _leaky_relu(q + jnp.sum(c0_ref[:, :] * ca2_ref[0, :][None, :], axis=1))
    e2 = _leaky_relu(q + jnp.sum(c1_ref[:, :] * ca2_ref[0, :][None, :], axis=1))
    m = jnp.maximum(jnp.maximum(e0, e1), e2)
    p0 = jnp.exp(e0 - m)
    p1 = jnp.exp(e1 - m)
    p2 = jnp.exp(e2 - m)
    s = p0 + p1 + p2
    pooled = (p0[:, None] * cls[None, :] + p1[:, None] * c0_ref[:, :]
              + p2[:, None] * c1_ref[:, :]) / s[:, None]
    out = jnp.dot(pooled, dw_ref[:, :], preferred_element_type=jnp.float32)
    out_ref[:, :] = _elu(out + db_ref[0, :][None, :])


def kernel(x, adjs, embedding_weight, W0, a0, cls_weight, cls_a, dense_W,
           dense_b):
    k, b, n = x.shape
    heads, embed, hidden = W0.shape
    d = heads * hidden
    pen = dense_W.shape[1]

    x_r = x.reshape(k * b, 1, n)
    adj_r = adjs.reshape(k * b, n, n)
    w_cat = jnp.transpose(W0, (1, 0, 2)).reshape(embed, d)
    a1s = a0[:, :hidden, 0]
    a2s = a0[:, hidden:, 0]
    v1 = jnp.einsum('hef,hf->eh', W0, a1s)
    v2 = jnp.einsum('hef,hf->he', W0, a2s)
    emb_t = embedding_weight.T

    codes = pl.pallas_call(
        functools.partial(_gat_block_kernel, heads=heads, hidden=hidden),
        grid=(k * b,),
        in_specs=[
            pl.BlockSpec((1, 1, n), lambda i: (i, 0, 0)),
            pl.BlockSpec((1, n, n), lambda i: (i, 0, 0)),
            pl.BlockSpec((n, embed), lambda i: (0, 0)),
            pl.BlockSpec((embed, n), lambda i: (0, 0)),
            pl.BlockSpec((embed, d), lambda i: (0, 0)),
            pl.BlockSpec((embed, heads), lambda i: (0, 0)),
            pl.BlockSpec((heads, embed), lambda i: (0, 0)),
        ],
        out_specs=pl.BlockSpec((1, 1, d), lambda i: (i, 0, 0)),
        out_shape=jax.ShapeDtypeStruct((k * b, 1, d), jnp.float32),
        compiler_params=pltpu.CompilerParams(
            dimension_semantics=("parallel",)),
    )(x_r, adj_r, embedding_weight, emb_t, w_cat, v1, v2)

    codes = codes.reshape(k, b, d)
    out = pl.pallas_call(
        _tail_kernel,
        out_shape=jax.ShapeDtypeStruct((b, pen), jnp.float32),
    )(codes[0], codes[1], cls_weight.reshape(1, d),
      cls_a[:d, 0].reshape(1, d), cls_a[d:, 0].reshape(1, d),
      dense_W, dense_b.reshape(1, pen))
    return out


# speedup vs baseline: 6.9818x; 1.1510x over previous
import functools

import jax
import jax.numpy as jnp
from jax.experimental import pallas as pl
from jax.experimental.pallas import tpu as pltpu

ALPHA = 0.2
NEG_BIG = -9e15


def _leaky_relu(v):
    return jnp.where(v >= 0, v, ALPHA * v)


def _elu(v):
    return jnp.where(v > 0, v, jnp.exp(v) - 1.0)


def _gat_block_kernel(x_ref, adj_ref, emb_ref, embt_ref, w_ref, v1_ref,
                      v2_ref, out_ref, *, heads, hidden, pairs):
    n = adj_ref.shape[1]
    ones_col = jnp.ones((n, 1), dtype=jnp.bfloat16)
    for j in range(pairs):
        xv = x_ref[j, 0, :]
        adj = adj_ref[j, :, :]
        maskb = jnp.where(adj > 0, 1.0, 0.0).astype(jnp.bfloat16)
        feats = xv[:, None] * emb_ref[:, :]
        feats_t = embt_ref[:, :] * xv[None, :]
        wh_all = jnp.dot(feats, w_ref[:, :],
                         preferred_element_type=jnp.float32)
        e1c = jnp.dot(feats, v1_ref[:, :], preferred_element_type=jnp.float32)
        e2r = jnp.dot(v2_ref[:, :], feats_t,
                      preferred_element_type=jnp.float32)
        rcol = jnp.exp((ALPHA - 1.0) * e1c).astype(jnp.bfloat16)
        exp_e2 = jnp.exp(e2r).astype(jnp.bfloat16)
        exp_a2 = jnp.exp(ALPHA * e2r).astype(jnp.bfloat16)
        code = None
        for h in range(heads):
            ua = rcol[:, h:h + 1] * exp_a2[h:h + 1, :]
            p = jnp.maximum(exp_e2[h:h + 1, :], ua) * maskb
            wh1 = jnp.concatenate(
                [wh_all[:, h * hidden:(h + 1) * hidden].astype(jnp.bfloat16),
                 ones_col], axis=1)
            hs = jnp.dot(p, wh1, preferred_element_type=jnp.float32)
            inv = 1.0 / hs[:, hidden:hidden + 1]
            hp = _elu(hs[:, :hidden] * inv)
            gated = xv[:, None] * hp
            cpart = jnp.max(gated, axis=0)
            code = cpart if code is None else jnp.concatenate([code, cpart])
        out_ref[j, 0, :] = code


def _tail_kernel(c0_ref, c1_ref, cls_ref, ca1_ref, ca2_ref, dw_ref, db_ref,
                 out_ref):
    cls = cls_ref[0, :]
    q = jnp.sum(cls * ca1_ref[0, :])
    e0 = _leaky_relu(q + jnp.sum(cls * ca2_ref[0, :]))
    e1 = _leaky_relu(q + jnp.sum(c0_ref[:, :] * ca2_ref[0, :][None, :], axis=1))
    e2 = _leaky_relu(q + jnp.sum(c1_ref[:, :] * ca2_ref[0, :][None, :], axis=1))
    m = jnp.maximum(jnp.maximum(e0, e1), e2)
    p0 = jnp.exp(e0 - m)
    p1 = jnp.exp(e1 - m)
    p2 = jnp.exp(e2 - m)
    s = p0 + p1 + p2
    pooled = (p0[:, None] * cls[None, :] + p1[:, None] * c0_ref[:, :]
              + p2[:, None] * c1_ref[:, :]) / s[:, None]
    out = jnp.dot(pooled, dw_ref[:, :], preferred_element_type=jnp.float32)
    out_ref[:, :] = _elu(out + db_ref[0, :][None, :])


def kernel(x, adjs, embedding_weight, W0, a0, cls_weight, cls_a, dense_W,
           dense_b):
    k, b, n = x.shape
    heads, embed, hidden = W0.shape
    d = heads * hidden
    pen = dense_W.shape[1]

    x_r = x.reshape(k * b, 1, n)
    adj_r = adjs.reshape(k * b, n, n)
    w_cat = jnp.transpose(W0, (1, 0, 2)).reshape(embed, d)
    a1s = a0[:, :hidden, 0]
    a2s = a0[:, hidden:, 0]
    v1 = jnp.einsum('hef,hf->eh', W0, a1s)
    v2 = jnp.einsum('hef,hf->he', W0, a2s)
    emb_t = embedding_weight.T

    pairs = 2
    codes = pl.pallas_call(
        functools.partial(_gat_block_kernel, heads=heads, hidden=hidden,
                          pairs=pairs),
        grid=(k * b // pairs,),
        in_specs=[
            pl.BlockSpec((pairs, 1, n), lambda i: (i, 0, 0)),
            pl.BlockSpec((pairs, n, n), lambda i: (i, 0, 0)),
            pl.BlockSpec((n, embed), lambda i: (0, 0)),
            pl.BlockSpec((embed, n), lambda i: (0, 0)),
            pl.BlockSpec((embed, d), lambda i: (0, 0)),
            pl.BlockSpec((embed, heads), lambda i: (0, 0)),
            pl.BlockSpec((heads, embed), lambda i: (0, 0)),
        ],
        out_specs=pl.BlockSpec((pairs, 1, d), lambda i: (i, 0, 0)),
        out_shape=jax.ShapeDtypeStruct((k * b, 1, d), jnp.float32),
        compiler_params=pltpu.CompilerParams(
            dimension_semantics=("parallel",)),
    )(x_r, adj_r, embedding_weight, emb_t, w_cat, v1, v2)

    codes = codes.reshape(k, b, d)
    out = pl.pallas_call(
        _tail_kernel,
        out_shape=jax.ShapeDtypeStruct((b, pen), jnp.float32),
    )(codes[0], codes[1], cls_weight.reshape(1, d),
      cls_a[:d, 0].reshape(1, d), cls_a[d:, 0].reshape(1, d),
      dense_W, dense_b.reshape(1, pen))
    return out


# speedup vs baseline: 7.2344x; 1.0362x over previous
import functools

import jax
import jax.numpy as jnp
from jax.experimental import pallas as pl
from jax.experimental.pallas import tpu as pltpu

ALPHA = 0.2
NEG_BIG = -9e15


def _leaky_relu(v):
    return jnp.where(v >= 0, v, ALPHA * v)


def _elu(v):
    return jnp.where(v > 0, v, jnp.exp(v) - 1.0)


def _gat_block_kernel(x_ref, adj_ref, emb_ref, embt_ref, w_ref, v1_ref,
                      v2_ref, out_ref, *, heads, hidden, pairs):
    n = adj_ref.shape[1]
    ones_col = jnp.ones((n, 1), dtype=jnp.bfloat16)
    for j in range(pairs):
        xv = x_ref[j, 0, :]
        adj = adj_ref[j, :, :]
        maskb = jnp.where(adj > 0, 1.0, 0.0).astype(jnp.bfloat16)
        feats = xv[:, None] * emb_ref[:, :]
        feats_t = embt_ref[:, :] * xv[None, :]
        wh_all = jnp.dot(feats, w_ref[:, :],
                         preferred_element_type=jnp.float32)
        e1c = jnp.dot(feats, v1_ref[:, :], preferred_element_type=jnp.float32)
        e2r = jnp.dot(v2_ref[:, :], feats_t,
                      preferred_element_type=jnp.float32)
        rcol = jnp.exp((ALPHA - 1.0) * e1c).astype(jnp.bfloat16)
        exp_e2 = jnp.exp(e2r).astype(jnp.bfloat16)
        exp_a2 = jnp.exp(ALPHA * e2r).astype(jnp.bfloat16)
        code = None
        for h in range(heads):
            ua = rcol[:, h:h + 1] * exp_a2[h:h + 1, :]
            p = jnp.maximum(exp_e2[h:h + 1, :], ua) * maskb
            wh1 = jnp.concatenate(
                [wh_all[:, h * hidden:(h + 1) * hidden].astype(jnp.bfloat16),
                 ones_col], axis=1)
            hs = jnp.dot(p, wh1, preferred_element_type=jnp.float32)
            inv = 1.0 / hs[:, hidden:hidden + 1]
            hp = _elu(hs[:, :hidden] * inv)
            gated = xv[:, None] * hp
            cpart = jnp.max(gated, axis=0)
            code = cpart if code is None else jnp.concatenate([code, cpart])
        out_ref[j, 0, :] = code


def _tail_kernel(c0_ref, c1_ref, cls_ref, ca1_ref, ca2_ref, dw_ref, db_ref,
                 out_ref):
    cls = cls_ref[0, :]
    q = jnp.sum(cls * ca1_ref[0, :])
    e0 = _leaky_relu(q + jnp.sum(cls * ca2_ref[0, :]))
    e1 = _leaky_relu(q + jnp.sum(c0_ref[:, :] * ca2_ref[0, :][None, :], axis=1))
    e2 = _leaky_relu(q + jnp.sum(c1_ref[:, :] * ca2_ref[0, :][None, :], axis=1))
    m = jnp.maximum(jnp.maximum(e0, e1), e2)
    p0 = jnp.exp(e0 - m)
    p1 = jnp.exp(e1 - m)
    p2 = jnp.exp(e2 - m)
    s = p0 + p1 + p2
    pooled = (p0[:, None] * cls[None, :] + p1[:, None] * c0_ref[:, :]
              + p2[:, None] * c1_ref[:, :]) / s[:, None]
    out = jnp.dot(pooled, dw_ref[:, :], preferred_element_type=jnp.float32)
    out_ref[:, :] = _elu(out + db_ref[0, :][None, :])


def kernel(x, adjs, embedding_weight, W0, a0, cls_weight, cls_a, dense_W,
           dense_b):
    k, b, n = x.shape
    heads, embed, hidden = W0.shape
    d = heads * hidden
    pen = dense_W.shape[1]

    x_r = x.reshape(k * b, 1, n)
    adj_r = adjs.reshape(k * b, n, n)
    w_cat = jnp.transpose(W0, (1, 0, 2)).reshape(embed, d)
    a1s = a0[:, :hidden, 0]
    a2s = a0[:, hidden:, 0]
    v1 = jnp.einsum('hef,hf->eh', W0, a1s)
    v2 = jnp.einsum('hef,hf->he', W0, a2s)
    emb_t = embedding_weight.T

    pairs = 4
    codes = pl.pallas_call(
        functools.partial(_gat_block_kernel, heads=heads, hidden=hidden,
                          pairs=pairs),
        grid=(k * b // pairs,),
        in_specs=[
            pl.BlockSpec((pairs, 1, n), lambda i: (i, 0, 0)),
            pl.BlockSpec((pairs, n, n), lambda i: (i, 0, 0)),
            pl.BlockSpec((n, embed), lambda i: (0, 0)),
            pl.BlockSpec((embed, n), lambda i: (0, 0)),
            pl.BlockSpec((embed, d), lambda i: (0, 0)),
            pl.BlockSpec((embed, heads), lambda i: (0, 0)),
            pl.BlockSpec((heads, embed), lambda i: (0, 0)),
        ],
        out_specs=pl.BlockSpec((pairs, 1, d), lambda i: (i, 0, 0)),
        out_shape=jax.ShapeDtypeStruct((k * b, 1, d), jnp.float32),
        compiler_params=pltpu.CompilerParams(
            dimension_semantics=("parallel",)),
    )(x_r, adj_r, embedding_weight, emb_t, w_cat, v1, v2)

    codes = codes.reshape(k, b, d)
    out = pl.pallas_call(
        _tail_kernel,
        out_shape=jax.ShapeDtypeStruct((b, pen), jnp.float32),
    )(codes[0], codes[1], cls_weight.reshape(1, d),
      cls_a[:d, 0].reshape(1, d), cls_a[d:, 0].reshape(1, d),
      dense_W, dense_b.reshape(1, pen))
    return out


# speedup vs baseline: 7.2874x; 1.0073x over previous
import functools

import jax
import jax.numpy as jnp
from jax.experimental import pallas as pl
from jax.experimental.pallas import tpu as pltpu

ALPHA = 0.2
NEG_BIG = -9e15


def _leaky_relu(v):
    return jnp.where(v >= 0, v, ALPHA * v)


def _elu(v):
    return jnp.where(v > 0, v, jnp.exp(v) - 1.0)


def _gat_block_kernel(x_ref, adj_ref, emb_ref, embt_ref, w_ref, v1_ref,
                      v2_ref, out_ref, *, heads, hidden, pairs):
    n = adj_ref.shape[1]
    ones_col = jnp.ones((n, 1), dtype=jnp.bfloat16)
    for j in range(pairs):
        xv = x_ref[j, 0, :]
        adj = adj_ref[j, :, :]
        maskb = jnp.where(adj > 0, 1.0, 0.0).astype(jnp.bfloat16)
        feats = xv[:, None] * emb_ref[:, :]
        feats_t = embt_ref[:, :] * xv[None, :]
        wh_all = jnp.dot(feats, w_ref[:, :],
                         preferred_element_type=jnp.float32)
        e1c = jnp.dot(feats, v1_ref[:, :], preferred_element_type=jnp.float32)
        e2r = jnp.dot(v2_ref[:, :], feats_t,
                      preferred_element_type=jnp.float32)
        rcol = jnp.exp((ALPHA - 1.0) * e1c).astype(jnp.bfloat16)
        exp_e2 = jnp.exp(e2r).astype(jnp.bfloat16)
        exp_a2 = jnp.exp(ALPHA * e2r).astype(jnp.bfloat16)
        code = None
        for h in range(heads):
            ua = rcol[:, h:h + 1] * exp_a2[h:h + 1, :]
            p = jnp.maximum(exp_e2[h:h + 1, :], ua) * maskb
            wh1 = jnp.concatenate(
                [wh_all[:, h * hidden:(h + 1) * hidden].astype(jnp.bfloat16),
                 ones_col], axis=1)
            hs = jnp.dot(p, wh1, preferred_element_type=jnp.float32)
            inv = 1.0 / hs[:, hidden:hidden + 1]
            hp = _elu(hs[:, :hidden] * inv)
            gated = xv[:, None] * hp
            cpart = jnp.max(gated, axis=0)
            code = cpart if code is None else jnp.concatenate([code, cpart])
        out_ref[j, 0, :] = code


def _tail_kernel(c0_ref, c1_ref, cls_ref, ca1_ref, ca2_ref, dw_ref, db_ref,
                 out_ref):
    cls = cls_ref[0, :]
    q = jnp.sum(cls * ca1_ref[0, :])
    e0 = _leaky_relu(q + jnp.sum(cls * ca2_ref[0, :]))
    e1 = _leaky_relu(q + jnp.sum(c0_ref[:, :] * ca2_ref[0, :][None, :], axis=1))
    e2 = _leaky_relu(q + jnp.sum(c1_ref[:, :] * ca2_ref[0, :][None, :], axis=1))
    m = jnp.maximum(jnp.maximum(e0, e1), e2)
    p0 = jnp.exp(e0 - m)
    p1 = jnp.exp(e1 - m)
    p2 = jnp.exp(e2 - m)
    s = p0 + p1 + p2
    pooled = (p0[:, None] * cls[None, :] + p1[:, None] * c0_ref[:, :]
              + p2[:, None] * c1_ref[:, :]) / s[:, None]
    out = jnp.dot(pooled, dw_ref[:, :], preferred_element_type=jnp.float32)
    out_ref[:, :] = _elu(out + db_ref[0, :][None, :])


def kernel(x, adjs, embedding_weight, W0, a0, cls_weight, cls_a, dense_W,
           dense_b):
    k, b, n = x.shape
    heads, embed, hidden = W0.shape
    d = heads * hidden
    pen = dense_W.shape[1]

    x_r = x.reshape(k * b, 1, n)
    adj_r = adjs.reshape(k * b, n, n)
    w_cat = jnp.transpose(W0, (1, 0, 2)).reshape(embed, d)
    a1s = a0[:, :hidden, 0]
    a2s = a0[:, hidden:, 0]
    v1 = jnp.einsum('hef,hf->eh', W0, a1s)
    v2 = jnp.einsum('hef,hf->he', W0, a2s)
    emb_t = embedding_weight.T

    pairs = 8
    codes = pl.pallas_call(
        functools.partial(_gat_block_kernel, heads=heads, hidden=hidden,
                          pairs=pairs),
        grid=(k * b // pairs,),
        in_specs=[
            pl.BlockSpec((pairs, 1, n), lambda i: (i, 0, 0)),
            pl.BlockSpec((pairs, n, n), lambda i: (i, 0, 0)),
            pl.BlockSpec((n, embed), lambda i: (0, 0)),
            pl.BlockSpec((embed, n), lambda i: (0, 0)),
            pl.BlockSpec((embed, d), lambda i: (0, 0)),
            pl.BlockSpec((embed, heads), lambda i: (0, 0)),
            pl.BlockSpec((heads, embed), lambda i: (0, 0)),
        ],
        out_specs=pl.BlockSpec((pairs, 1, d), lambda i: (i, 0, 0)),
        out_shape=jax.ShapeDtypeStruct((k * b, 1, d), jnp.float32),
        compiler_params=pltpu.CompilerParams(
            dimension_semantics=("parallel",)),
    )(x_r, adj_r, embedding_weight, emb_t, w_cat, v1, v2)

    codes = codes.reshape(k, b, d)
    out = pl.pallas_call(
        _tail_kernel,
        out_shape=jax.ShapeDtypeStruct((b, pen), jnp.float32),
    )(codes[0], codes[1], cls_weight.reshape(1, d),
      cls_a[:d, 0].reshape(1, d), cls_a[d:, 0].reshape(1, d),
      dense_W, dense_b.reshape(1, pen))
    return out


# speedup vs baseline: 7.6237x; 1.0462x over previous
import functools

import jax
import jax.numpy as jnp
from jax.experimental import pallas as pl
from jax.experimental.pallas import tpu as pltpu

ALPHA = 0.2
NEG_BIG = -9e15


def _leaky_relu(v):
    return jnp.where(v >= 0, v, ALPHA * v)


def _elu(v):
    return jnp.where(v > 0, v, jnp.exp(v) - 1.0)


def _gat_block_kernel(x_ref, adj_ref, emb_ref, embt_ref, w_ref, v1_ref,
                      v2_ref, out_ref, *, heads, hidden, pairs):
    n = adj_ref.shape[1]
    ones_col = jnp.ones((n, 1), dtype=jnp.bfloat16)
    for j in range(pairs):
        xv = x_ref[j, 0, :]
        xvb = xv.astype(jnp.bfloat16)
        adj = adj_ref[j, :, :]
        maskb = jnp.where(adj > 0, 1.0, 0.0).astype(jnp.bfloat16)
        feats = xv[:, None] * emb_ref[:, :]
        feats_t = embt_ref[:, :] * xv[None, :]
        wh_all = jnp.dot(feats, w_ref[:, :],
                         preferred_element_type=jnp.float32)
        e1c = jnp.dot(feats, v1_ref[:, :], preferred_element_type=jnp.float32)
        e2r = jnp.dot(v2_ref[:, :], feats_t,
                      preferred_element_type=jnp.float32)
        rcol = jnp.exp((ALPHA - 1.0) * e1c).astype(jnp.bfloat16)
        exp_e2 = jnp.exp(e2r).astype(jnp.bfloat16)
        exp_a2 = jnp.exp(ALPHA * e2r).astype(jnp.bfloat16)
        code = None
        for h in range(heads):
            ua = rcol[:, h:h + 1] * exp_a2[h:h + 1, :]
            p = jnp.maximum(exp_e2[h:h + 1, :], ua) * maskb
            wh1 = jnp.concatenate(
                [wh_all[:, h * hidden:(h + 1) * hidden].astype(jnp.bfloat16),
                 ones_col], axis=1)
            hs = jnp.dot(p, wh1, preferred_element_type=jnp.float32)
            inv = (1.0 / hs[:, hidden:hidden + 1]).astype(jnp.bfloat16)
            hpb = hs[:, :hidden].astype(jnp.bfloat16) * inv
            hp = jnp.where(hpb > 0, hpb, jnp.exp(hpb) - 1.0)
            gated = xvb[:, None] * hp
            cpart = jnp.max(gated, axis=0)
            code = cpart if code is None else jnp.concatenate([code, cpart])
        out_ref[j, 0, :] = code.astype(jnp.float32)


def _tail_kernel(c0_ref, c1_ref, cls_ref, ca1_ref, ca2_ref, dw_ref, db_ref,
                 out_ref):
    cls = cls_ref[0, :]
    q = jnp.sum(cls * ca1_ref[0, :])
    e0 = _leaky_relu(q + jnp.sum(cls * ca2_ref[0, :]))
    e1 = _leaky_relu(q + jnp.sum(c0_ref[:, :] * ca2_ref[0, :][None, :], axis=1))
    e2 = _leaky_relu(q + jnp.sum(c1_ref[:, :] * ca2_ref[0, :][None, :], axis=1))
    m = jnp.maximum(jnp.maximum(e0, e1), e2)
    p0 = jnp.exp(e0 - m)
    p1 = jnp.exp(e1 - m)
    p2 = jnp.exp(e2 - m)
    s = p0 + p1 + p2
    pooled = (p0[:, None] * cls[None, :] + p1[:, None] * c0_ref[:, :]
              + p2[:, None] * c1_ref[:, :]) / s[:, None]
    out = jnp.dot(pooled, dw_ref[:, :], preferred_element_type=jnp.float32)
    out_ref[:, :] = _elu(out + db_ref[0, :][None, :])


def kernel(x, adjs, embedding_weight, W0, a0, cls_weight, cls_a, dense_W,
           dense_b):
    k, b, n = x.shape
    heads, embed, hidden = W0.shape
    d = heads * hidden
    pen = dense_W.shape[1]

    x_r = x.reshape(k * b, 1, n)
    adj_r = adjs.reshape(k * b, n, n)
    w_cat = jnp.transpose(W0, (1, 0, 2)).reshape(embed, d)
    a1s = a0[:, :hidden, 0]
    a2s = a0[:, hidden:, 0]
    v1 = jnp.einsum('hef,hf->eh', W0, a1s)
    v2 = jnp.einsum('hef,hf->he', W0, a2s)
    emb_t = embedding_weight.T

    pairs = 8
    codes = pl.pallas_call(
        functools.partial(_gat_block_kernel, heads=heads, hidden=hidden,
                          pairs=pairs),
        grid=(k * b // pairs,),
        in_specs=[
            pl.BlockSpec((pairs, 1, n), lambda i: (i, 0, 0)),
            pl.BlockSpec((pairs, n, n), lambda i: (i, 0, 0)),
            pl.BlockSpec((n, embed), lambda i: (0, 0)),
            pl.BlockSpec((embed, n), lambda i: (0, 0)),
            pl.BlockSpec((embed, d), lambda i: (0, 0)),
            pl.BlockSpec((embed, heads), lambda i: (0, 0)),
            pl.BlockSpec((heads, embed), lambda i: (0, 0)),
        ],
        out_specs=pl.BlockSpec((pairs, 1, d), lambda i: (i, 0, 0)),
        out_shape=jax.ShapeDtypeStruct((k * b, 1, d), jnp.float32),
        compiler_params=pltpu.CompilerParams(
            dimension_semantics=("parallel",)),
    )(x_r, adj_r, embedding_weight, emb_t, w_cat, v1, v2)

    codes = codes.reshape(k, b, d)
    out = pl.pallas_call(
        _tail_kernel,
        out_shape=jax.ShapeDtypeStruct((b, pen), jnp.float32),
    )(codes[0], codes[1], cls_weight.reshape(1, d),
      cls_a[:d, 0].reshape(1, d), cls_a[d:, 0].reshape(1, d),
      dense_W, dense_b.reshape(1, pen))
    return out


# speedup vs baseline: 9.0042x; 1.1811x over previous
import functools

import jax
import jax.numpy as jnp
from jax.experimental import pallas as pl
from jax.experimental.pallas import tpu as pltpu

ALPHA = 0.2
NEG_BIG = -9e15


def _leaky_relu(v):
    return jnp.where(v >= 0, v, ALPHA * v)


def _elu(v):
    return jnp.where(v > 0, v, jnp.exp(v) - 1.0)


def _gat_block_kernel(x_ref, adj_ref, g_ref, ev1_ref, ev2t_ref, out_ref, *,
                      heads, hidden, pairs):
    n = adj_ref.shape[1]
    ones_col = jnp.ones((n, 1), dtype=jnp.bfloat16)
    for j in range(pairs):
        xv = x_ref[j, 0, :]
        xvb = xv.astype(jnp.bfloat16)
        adj = adj_ref[j, :, :]
        maskb = jnp.where(adj > 0, 1.0, 0.0).astype(jnp.bfloat16)
        e1c = xv[:, None] * ev1_ref[:, :]
        e2r = ev2t_ref[:, :] * xv[None, :]
        rcol = jnp.exp((ALPHA - 1.0) * e1c).astype(jnp.bfloat16)
        exp_e2 = jnp.exp(e2r).astype(jnp.bfloat16)
        exp_a2 = jnp.exp(ALPHA * e2r).astype(jnp.bfloat16)
        gw = xvb[:, None] * g_ref[:, :]
        code = None
        for h in range(heads):
            ua = rcol[:, h:h + 1] * exp_a2[h:h + 1, :]
            p = jnp.maximum(exp_e2[h:h + 1, :], ua) * maskb
            wh1 = jnp.concatenate(
                [gw[:, h * hidden:(h + 1) * hidden], ones_col], axis=1)
            hs = jnp.dot(p, wh1, preferred_element_type=jnp.float32)
            inv = (1.0 / hs[:, hidden:hidden + 1]).astype(jnp.bfloat16)
            hpb = hs[:, :hidden].astype(jnp.bfloat16) * inv
            hp = jnp.where(hpb > 0, hpb, jnp.exp(hpb) - 1.0)
            gated = xvb[:, None] * hp
            cpart = jnp.max(gated, axis=0)
            code = cpart if code is None else jnp.concatenate([code, cpart])
        out_ref[j, 0, :] = code.astype(jnp.float32)


def _tail_kernel(c0_ref, c1_ref, cls_ref, ca1_ref, ca2_ref, dw_ref, db_ref,
                 out_ref):
    cls = cls_ref[0, :]
    q = jnp.sum(cls * ca1_ref[0, :])
    e0 = _leaky_relu(q + jnp.sum(cls * ca2_ref[0, :]))
    e1 = _leaky_relu(q + jnp.sum(c0_ref[:, :] * ca2_ref[0, :][None, :], axis=1))
    e2 = _leaky_relu(q + jnp.sum(c1_ref[:, :] * ca2_ref[0, :][None, :], axis=1))
    m = jnp.maximum(jnp.maximum(e0, e1), e2)
    p0 = jnp.exp(e0 - m)
    p1 = jnp.exp(e1 - m)
    p2 = jnp.exp(e2 - m)
    s = p0 + p1 + p2
    pooled = (p0[:, None] * cls[None, :] + p1[:, None] * c0_ref[:, :]
              + p2[:, None] * c1_ref[:, :]) / s[:, None]
    out = jnp.dot(pooled, dw_ref[:, :], preferred_element_type=jnp.float32)
    out_ref[:, :] = _elu(out + db_ref[0, :][None, :])


def kernel(x, adjs, embedding_weight, W0, a0, cls_weight, cls_a, dense_W,
           dense_b):
    k, b, n = x.shape
    heads, embed, hidden = W0.shape
    d = heads * hidden
    pen = dense_W.shape[1]

    x_r = x.reshape(k * b, 1, n)
    adj_r = adjs.reshape(k * b, n, n)
    w_cat = jnp.transpose(W0, (1, 0, 2)).reshape(embed, d)
    a1s = a0[:, :hidden, 0]
    a2s = a0[:, hidden:, 0]
    v1 = jnp.einsum('hef,hf->eh', W0, a1s)
    v2 = jnp.einsum('hef,hf->he', W0, a2s)
    g_bf = (embedding_weight @ w_cat).astype(jnp.bfloat16)
    ev1 = embedding_weight @ v1
    ev2t = (embedding_weight @ v2.T).T

    pairs = 8
    codes = pl.pallas_call(
        functools.partial(_gat_block_kernel, heads=heads, hidden=hidden,
                          pairs=pairs),
        grid=(k * b // pairs,),
        in_specs=[
            pl.BlockSpec((pairs, 1, n), lambda i: (i, 0, 0)),
            pl.BlockSpec((pairs, n, n), lambda i: (i, 0, 0)),
            pl.BlockSpec((n, d), lambda i: (0, 0)),
            pl.BlockSpec((n, heads), lambda i: (0, 0)),
            pl.BlockSpec((heads, n), lambda i: (0, 0)),
        ],
        out_specs=pl.BlockSpec((pairs, 1, d), lambda i: (i, 0, 0)),
        out_shape=jax.ShapeDtypeStruct((k * b, 1, d), jnp.float32),
        compiler_params=pltpu.CompilerParams(
            dimension_semantics=("parallel",)),
    )(x_r, adj_r, g_bf, ev1, ev2t)

    codes = codes.reshape(k, b, d)
    out = pl.pallas_call(
        _tail_kernel,
        out_shape=jax.ShapeDtypeStruct((b, pen), jnp.float32),
    )(codes[0], codes[1], cls_weight.reshape(1, d),
      cls_a[:d, 0].reshape(1, d), cls_a[d:, 0].reshape(1, d),
      dense_W, dense_b.reshape(1, pen))
    return out
